# baseline (speedup 1.0000x reference)
import functools

import jax
import jax.numpy as jnp
from jax import lax
from jax.experimental import pallas as pl
from jax.experimental.pallas import tpu as pltpu
from jax.experimental.pallas import tpu_sc as plsc

_VOCAB = 1000000
_D = 64
_NEG = 5
_B = 16384

_NC = 2
_NS = 16
_NW = _NC * _NS
_ROWS_PER_TILE = _B // _NW
_C = 64
_NCHUNK = _ROWS_PER_TILE // _C
_GROUPS = _C // 16


def _softplus(z):
    t = jnp.exp(-jnp.abs(z))
    s = t / (t + 2.0)
    s2 = s * s
    p = jnp.float32(1.0 / 9.0)
    p = p * s2 + jnp.float32(1.0 / 7.0)
    p = p * s2 + jnp.float32(1.0 / 5.0)
    p = p * s2 + jnp.float32(1.0 / 3.0)
    p = p * s2 + jnp.float32(1.0)
    return jnp.maximum(z, 0.0) + 2.0 * s * p


def _sc_body(widxp_hbm, e1idxp_hbm, hpack_hbm, mask_hbm,
             emb0_hbm, emb1_hbm, out_hbm,
             idx0p_v, idx1p_v, hpack_v, mask_v, w_v, e1_v, out_v,
             sem_a, sem_b):
    wid = lax.axis_index("s") * _NC + lax.axis_index("c")

    pltpu.sync_copy(widxp_hbm.at[wid], idx0p_v)
    pltpu.sync_copy(e1idxp_hbm.at[wid], idx1p_v)
    pltpu.sync_copy(hpack_hbm.at[wid], hpack_v)

    sems = [sem_a, sem_b]

    def issue(c, buf):
        cps = [pltpu.async_copy(emb0_hbm.at[idx0p_v.at[c]], w_v.at[buf],
                                sems[buf]),
               pltpu.async_copy(mask_hbm.at[wid, c], mask_v.at[buf],
                                sems[buf])]
        for j in range(6):
            cps.append(pltpu.async_copy(emb1_hbm.at[idx1p_v.at[c * 6 + j]],
                                        e1_v.at[buf, j], sems[buf]))
        return cps

    lane = lax.iota(jnp.int32, 16)
    one = jnp.full((16,), 1, dtype=jnp.int32)

    def compute_chunk(c, buf, accs):
        wbuf = w_v.at[buf]
        ebuf = e1_v.at[buf]
        bvec = jnp.full((16,), buf, dtype=jnp.int32)

        def gbody(g, accs):
            acc_pos, acc_neg = accs
            rloc = lane + g * 16
            rt = rloc + c * _C
            hp = plsc.load_gather(hpack_v, [rt])
            hw = lax.shift_left(lax.bitwise_and(hp, one), 6)
            flat6 = rloc * 6
            js = []
            iss = []
            hks = []
            for k in range(6):
                f = flat6 + k
                js.append(lax.shift_right_logical(f, 6))
                iss.append(lax.bitwise_and(f, _C - 1))
                hks.append(lax.shift_left(
                    lax.bitwise_and(
                        lax.shift_right_logical(hp, k + 1), one), 6))

            def dbody(d, dots):
                dvec = jnp.full((16,), d, dtype=jnp.int32)
                wv = plsc.load_gather(wbuf, [rloc, hw + dvec])
                cv = plsc.load_gather(ebuf, [js[0], iss[0], hks[0] + dvec])
                new = [dots[0] + wv * cv]
                for k in range(1, 6):
                    nv = plsc.load_gather(ebuf, [js[k], iss[k], hks[k] + dvec])
                    new.append(dots[k] + nv * wv)
                return tuple(new)

            zero = jnp.zeros((16,), jnp.float32)
            dots = lax.fori_loop(0, _D, dbody, (zero,) * 6)

            acc_pos = acc_pos + _softplus(-dots[0])
            for k in range(1, 6):
                mvec = plsc.load_gather(mask_v, [bvec, rloc * _NEG + (k - 1)])
                acc_neg = acc_neg + _softplus(dots[k] * mvec)
            return (acc_pos, acc_neg)

        return lax.fori_loop(0, _GROUPS, gbody, accs)

    zero = jnp.zeros((16,), jnp.float32)
    accs = (zero, zero)
    descs = [None, None]
    descs[0] = issue(0, 0)
    for c in range(_NCHUNK):
        buf = c % 2
        if c + 1 < _NCHUNK:
            descs[(c + 1) % 2] = issue(c + 1, (c + 1) % 2)
        for d in descs[buf]:
            d.wait()
        accs = compute_chunk(c, buf, accs)

    out_v[0, :] = accs[0]
    out_v[1, :] = accs[1]
    pltpu.sync_copy(out_v, out_hbm.at[wid])


_mesh = plsc.VectorSubcoreMesh(core_axis_name="c", subcore_axis_name="s",
                               num_cores=_NC, num_subcores=_NS)

_sc_kernel = functools.partial(
    pl.kernel,
    out_type=jax.ShapeDtypeStruct((_NW, 2, 16), jnp.float32),
    mesh=_mesh,
    compiler_params=pltpu.CompilerParams(needs_layout_passes=False,
                                         use_tc_tiling_on_sc=True),
    scratch_types=[
        pltpu.VMEM((_NCHUNK, _C), jnp.int32),
        pltpu.VMEM((_NCHUNK * 6, _C), jnp.int32),
        pltpu.VMEM((_ROWS_PER_TILE,), jnp.int32),
        pltpu.VMEM((2, _NEG * _C), jnp.float32),
        pltpu.VMEM((2, _C, 128), jnp.float32),
        pltpu.VMEM((2, 6, _C, 128), jnp.float32),
        pltpu.VMEM((2, 16), jnp.float32),
        pltpu.SemaphoreType.DMA,
        pltpu.SemaphoreType.DMA,
    ],
)(_sc_body)


_TROWS = 16384
_SPLIT = 30 * _TROWS
_RPAD = 1000064
_PROWS = _RPAD - _SPLIT


def _tc_pack_body(lo_ref, hi_ref, out_ref):
    out_ref[...] = jnp.concatenate([lo_ref[...].T, hi_ref[...].T], axis=1)


_tc_pack = pl.pallas_call(
    _tc_pack_body,
    grid=((_PROWS + _TROWS - 1) // _TROWS,),
    in_specs=[pl.BlockSpec((_D, _TROWS), lambda j: (0, j)),
              pl.BlockSpec((_D, _TROWS), lambda j: (0, j + _SPLIT // _TROWS))],
    out_specs=pl.BlockSpec((_TROWS, 2 * _D), lambda j: (j, 0)),
    out_shape=jax.ShapeDtypeStruct((_PROWS, 2 * _D), jnp.float32),
)


_PAIRJ = _SPLIT // 128
_NBLK = _RPAD // 128
_GEN = _PAIRJ + (_NBLK - 2 * _PAIRJ) - 1
_ROUNDS = _GEN // _NW


def _sc_pack_body(src_hbm, tail_hbm, dst_hbm, in_v, out_v, tail_v,
                  sem_a, sem_b, sem_o):
    wid = lax.axis_index("s") * _NC + lax.axis_index("c")
    sems = [sem_a, sem_b]
    lane = lax.iota(jnp.int32, 16)
    rowv = [lane + 16 * v for v in range(8)]

    def issue(it, buf):
        for g in range(2):
            j = it if g == 0 else it + _PAIRJ
            for k in range(8):
                pltpu.async_copy(src_hbm.at[k, :, pl.ds(j * 128, 128)],
                                 in_v.at[buf, g * 8 + k], sems[buf])

    def wait_in(buf):
        for g in range(2):
            for k in range(8):
                pltpu.make_async_copy(src_hbm.at[k, :, pl.ds(0, 128)],
                                      in_v.at[buf, g * 8 + k],
                                      sems[buf]).wait()

    def transpose_item(buf):
        bvec = jnp.full((16,), buf, dtype=jnp.int32)
        for g in range(2):
            for k in range(8):
                blk = g * 8 + k
                cbase = g * 64 + 8 * k

                def abody(a, _):
                    cvec = jnp.full((16,), cbase, dtype=jnp.int32) + a
                    for v in range(8):
                        x = in_v[buf, blk, a, pl.ds(16 * v, 16)]
                        plsc.store_scatter(out_v, [bvec, rowv[v], cvec], x)
                    return 0

                lax.fori_loop(0, 8, abody, 0)

    def flush(it, buf):
        pltpu.async_copy(out_v.at[buf], dst_hbm.at[pl.ds(it * 128, 128)],
                         sem_o)

    def item_id(i):
        return jnp.minimum(i * _NW + wid, _GEN - 1)

    issue(item_id(0), 0)

    def body2(i2, _):
        for b in range(2):
            i = i2 * 2 + b
            wait_in(b)
            issue(item_id(i + 1), 1 - b)
            transpose_item(b)
            flush(item_id(i), b)
            pltpu.make_async_copy(out_v.at[b], dst_hbm.at[pl.ds(0, 128)],
                                  sem_o).wait()
        return 0

    lax.fori_loop(0, _ROUNDS // 2, body2, 0)
    wait_in(0)
    transpose_item(0)
    flush(item_id(_ROUNDS), 0)
    pltpu.make_async_copy(out_v.at[0], dst_hbm.at[pl.ds(0, 128)],
                          sem_o).wait()

    @pl.when(wid == _NW - 1)
    def _tail():
        pltpu.sync_copy(tail_hbm, tail_v)
        pltpu.sync_copy(tail_v, dst_hbm.at[pl.ds(_GEN * 128, 64)])


_sc_pack = functools.partial(
    pl.kernel,
    out_type=jax.ShapeDtypeStruct((_PROWS, 2 * _D), jnp.float32),
    mesh=_mesh,
    compiler_params=pltpu.CompilerParams(needs_layout_passes=False,
                                         use_tc_tiling_on_sc=True),
    scratch_types=[
        pltpu.VMEM((2, 16, 8, 128), jnp.float32),
        pltpu.VMEM((2, 128, 128), jnp.float32),
        pltpu.VMEM((64, 128), jnp.float32),
        pltpu.SemaphoreType.DMA,
        pltpu.SemaphoreType.DMA,
        pltpu.SemaphoreType.DMA,
    ],
)(_sc_pack_body)


def kernel(data, emb0, emb1):
    data = data.astype(jnp.int32)
    widx = data[:, 0]
    e1idx = data[:, 1:2 + _NEG]
    wh = (widx >= _SPLIT).astype(jnp.int32)
    e1h = (e1idx >= _SPLIT).astype(jnp.int32)
    widxp = (widx - wh * _SPLIT).reshape(_NW, _NCHUNK, _C)
    e1idxp = (e1idx - e1h * _SPLIT).reshape(_NW, _NCHUNK * 6, _C)
    hpack = (wh | jnp.left_shift(
        e1h, jnp.arange(1, 7, dtype=jnp.int32)[None, :]).sum(axis=1)
             ).reshape(_NW, _ROWS_PER_TILE)
    maskf = data[:, 2 + _NEG:].astype(jnp.float32).reshape(
        _NW, _NCHUNK, _NEG * _C)
    emb1t = emb1.T
    tailblk = jnp.concatenate(
        [jnp.zeros((64, _D), jnp.float32), emb0[_VOCAB - 64:]], axis=1)
    emb0p = _sc_pack(emb0.T.reshape(8, 8, _VOCAB), tailblk)
    emb1p = _tc_pack(emb1t, emb1t)
    out = _sc_kernel(widxp, e1idxp, hpack, maskf, emb0p, emb1p)
    pos_loss = jnp.sum(out[:, 0, :])
    neg_loss = jnp.sum(out[:, 1, :])
    return (pos_loss, neg_loss)


# speedup vs baseline: 2.4294x; 2.4294x over previous
import functools

import jax
import jax.numpy as jnp
from jax import lax
from jax.experimental import pallas as pl
from jax.experimental.pallas import tpu as pltpu
from jax.experimental.pallas import tpu_sc as plsc

_VOCAB = 1000000
_D = 64
_NEG = 5
_B = 16384

_NC = 2
_NS = 16
_NW = _NC * _NS
_ROWS_PER_TILE = _B // _NW
_C = 64
_NCHUNK = _ROWS_PER_TILE // _C
_GROUPS = _C // 16


def _softplus(z):
    t = jnp.exp(-jnp.abs(z))
    s = t / (t + 2.0)
    s2 = s * s
    p = jnp.float32(1.0 / 9.0)
    p = p * s2 + jnp.float32(1.0 / 7.0)
    p = p * s2 + jnp.float32(1.0 / 5.0)
    p = p * s2 + jnp.float32(1.0 / 3.0)
    p = p * s2 + jnp.float32(1.0)
    return jnp.maximum(z, 0.0) + 2.0 * s * p


def _sc_body(widxp_hbm, e1idxp_hbm, hpack_hbm, mask_hbm,
             emb0_hbm, emb1_hbm, out_hbm,
             idx0p_v, idx1p_v, hpack_v, mask_v, w_v, e1_v, out_v,
             sem_a, sem_b):
    wid = lax.axis_index("s") * _NC + lax.axis_index("c")

    pltpu.sync_copy(widxp_hbm.at[wid], idx0p_v)
    pltpu.sync_copy(e1idxp_hbm.at[wid], idx1p_v)
    pltpu.sync_copy(hpack_hbm.at[wid], hpack_v)

    sems = [sem_a, sem_b]

    def issue(c, buf):
        cps = [pltpu.async_copy(emb0_hbm.at[idx0p_v.at[c]], w_v.at[buf],
                                sems[buf]),
               pltpu.async_copy(mask_hbm.at[wid, c], mask_v.at[buf],
                                sems[buf])]
        for j in range(6):
            cps.append(pltpu.async_copy(emb1_hbm.at[idx1p_v.at[c * 6 + j]],
                                        e1_v.at[buf, j], sems[buf]))
        return cps

    lane = lax.iota(jnp.int32, 16)
    one = jnp.full((16,), 1, dtype=jnp.int32)

    def compute_chunk(c, buf, accs):
        wbuf = w_v.at[buf]
        ebuf = e1_v.at[buf]
        bvec = jnp.full((16,), buf, dtype=jnp.int32)

        def gbody(g, accs):
            acc_pos, acc_neg = accs
            rloc = lane + g * 16
            rt = rloc + c * _C
            hp = plsc.load_gather(hpack_v, [rt])
            hw = lax.shift_left(lax.bitwise_and(hp, one), 6)
            flat6 = rloc * 6
            js = []
            iss = []
            hks = []
            for k in range(6):
                f = flat6 + k
                js.append(lax.shift_right_logical(f, 6))
                iss.append(lax.bitwise_and(f, _C - 1))
                hks.append(lax.shift_left(
                    lax.bitwise_and(
                        lax.shift_right_logical(hp, k + 1), one), 6))

            def dbody(d, dots):
                dvec = jnp.full((16,), d, dtype=jnp.int32)
                wv = plsc.load_gather(wbuf, [rloc, hw + dvec])
                cv = plsc.load_gather(ebuf, [js[0], iss[0], hks[0] + dvec])
                new = [dots[0] + wv * cv]
                for k in range(1, 6):
                    nv = plsc.load_gather(ebuf, [js[k], iss[k], hks[k] + dvec])
                    new.append(dots[k] + nv * wv)
                return tuple(new)

            zero = jnp.zeros((16,), jnp.float32)
            dots = lax.fori_loop(0, _D, dbody, (zero,) * 6)

            acc_pos = acc_pos + _softplus(-dots[0])
            for k in range(1, 6):
                mvec = plsc.load_gather(mask_v, [bvec, rloc * _NEG + (k - 1)])
                acc_neg = acc_neg + _softplus(dots[k] * mvec)
            return (acc_pos, acc_neg)

        return lax.fori_loop(0, _GROUPS, gbody, accs)

    zero = jnp.zeros((16,), jnp.float32)
    accs = (zero, zero)
    descs = [None, None]
    descs[0] = issue(0, 0)
    for c in range(_NCHUNK):
        buf = c % 2
        if c + 1 < _NCHUNK:
            descs[(c + 1) % 2] = issue(c + 1, (c + 1) % 2)
        for d in descs[buf]:
            d.wait()
        accs = compute_chunk(c, buf, accs)

    out_v[0, :] = accs[0]
    out_v[1, :] = accs[1]
    pltpu.sync_copy(out_v, out_hbm.at[wid])


_mesh = plsc.VectorSubcoreMesh(core_axis_name="c", subcore_axis_name="s",
                               num_cores=_NC, num_subcores=_NS)

_sc_kernel = functools.partial(
    pl.kernel,
    out_type=jax.ShapeDtypeStruct((_NW, 2, 16), jnp.float32),
    mesh=_mesh,
    compiler_params=pltpu.CompilerParams(needs_layout_passes=False,
                                         use_tc_tiling_on_sc=True),
    scratch_types=[
        pltpu.VMEM((_NCHUNK, _C), jnp.int32),
        pltpu.VMEM((_NCHUNK * 6, _C), jnp.int32),
        pltpu.VMEM((_ROWS_PER_TILE,), jnp.int32),
        pltpu.VMEM((2, _NEG * _C), jnp.float32),
        pltpu.VMEM((2, _C, 128), jnp.float32),
        pltpu.VMEM((2, 6, _C, 128), jnp.float32),
        pltpu.VMEM((2, 16), jnp.float32),
        pltpu.SemaphoreType.DMA,
        pltpu.SemaphoreType.DMA,
    ],
)(_sc_body)


_TROWS = 16384
_SPLIT = 30 * _TROWS
_RPAD = 1000064
_PROWS = _RPAD - _SPLIT


def _tc_pack_body(lo_ref, hi_ref, out_ref):
    out_ref[...] = jnp.concatenate([lo_ref[...].T, hi_ref[...].T], axis=1)


_tc_pack = pl.pallas_call(
    _tc_pack_body,
    grid=((_PROWS + _TROWS - 1) // _TROWS,),
    in_specs=[pl.BlockSpec((_D, _TROWS), lambda j: (0, j)),
              pl.BlockSpec((_D, _TROWS), lambda j: (0, j + _SPLIT // _TROWS))],
    out_specs=pl.BlockSpec((_TROWS, 2 * _D), lambda j: (j, 0)),
    out_shape=jax.ShapeDtypeStruct((_PROWS, 2 * _D), jnp.float32),
)


def kernel(data, emb0, emb1):
    data = data.astype(jnp.int32)
    widx = data[:, 0]
    e1idx = data[:, 1:2 + _NEG]
    wh = (widx >= _SPLIT).astype(jnp.int32)
    e1h = (e1idx >= _SPLIT).astype(jnp.int32)
    widxp = (widx - wh * _SPLIT).reshape(_NW, _NCHUNK, _C)
    e1idxp = (e1idx - e1h * _SPLIT).reshape(_NW, _NCHUNK * 6, _C)
    hpack = (wh | jnp.left_shift(
        e1h, jnp.arange(1, 7, dtype=jnp.int32)[None, :]).sum(axis=1)
             ).reshape(_NW, _ROWS_PER_TILE)
    maskf = data[:, 2 + _NEG:].astype(jnp.float32).reshape(
        _NW, _NCHUNK, _NEG * _C)
    emb0t = emb0.T
    emb1t = emb1.T
    emb0p = _tc_pack(emb0t, emb0t)
    emb1p = _tc_pack(emb1t, emb1t)
    out = _sc_kernel(widxp, e1idxp, hpack, maskf, emb0p, emb1p)
    pos_loss = jnp.sum(out[:, 0, :])
    neg_loss = jnp.sum(out[:, 1, :])
    return (pos_loss, neg_loss)


# speedup vs baseline: 2.4903x; 1.0251x over previous
import functools

import jax
import jax.numpy as jnp
from jax import lax
from jax.experimental import pallas as pl
from jax.experimental.pallas import tpu as pltpu
from jax.experimental.pallas import tpu_sc as plsc

_VOCAB = 1000000
_D = 64
_NEG = 5
_B = 16384

_NC = 2
_NS = 16
_NW = _NC * _NS
_ROWS_PER_TILE = _B // _NW
_C = 64
_NCHUNK = _ROWS_PER_TILE // _C
_GROUPS = _C // 16


def _softplus(z):
    t = jnp.exp(-jnp.abs(z))
    s = t / (t + 2.0)
    s2 = s * s
    p = jnp.float32(1.0 / 9.0)
    p = p * s2 + jnp.float32(1.0 / 7.0)
    p = p * s2 + jnp.float32(1.0 / 5.0)
    p = p * s2 + jnp.float32(1.0 / 3.0)
    p = p * s2 + jnp.float32(1.0)
    return jnp.maximum(z, 0.0) + 2.0 * s * p


def _sc_body(widxp_hbm, e1idxp_hbm, hpack_hbm, mask_hbm,
             emb0_hbm, emb1_hbm, out_hbm,
             idx0p_v, idx1p_v, hpack_v, mask_v, w_v, e1_v, out_v,
             sem_a, sem_b):
    wid = lax.axis_index("s") * _NC + lax.axis_index("c")

    pltpu.sync_copy(widxp_hbm.at[wid], idx0p_v)
    pltpu.sync_copy(e1idxp_hbm.at[wid], idx1p_v)
    pltpu.sync_copy(hpack_hbm.at[wid], hpack_v)

    sems = [sem_a, sem_b]

    def issue(c, buf):
        cps = [pltpu.async_copy(emb0_hbm.at[idx0p_v.at[c]], w_v.at[buf],
                                sems[buf]),
               pltpu.async_copy(mask_hbm.at[wid, c], mask_v.at[buf],
                                sems[buf])]
        for j in range(6):
            cps.append(pltpu.async_copy(emb1_hbm.at[idx1p_v.at[c * 6 + j]],
                                        e1_v.at[buf, j], sems[buf]))
        return cps

    lane = lax.iota(jnp.int32, 16)
    one = jnp.full((16,), 1, dtype=jnp.int32)

    def compute_chunk(c, buf, accs):
        wbuf = w_v.at[buf]
        ebuf = e1_v.at[buf]
        bvec = jnp.full((16,), buf, dtype=jnp.int32)

        def gbody(g, accs):
            acc_pos, acc_neg = accs
            rloc = lane + g * 16
            rt = rloc + c * _C
            hp = plsc.load_gather(hpack_v, [rt])
            hw = lax.shift_left(lax.bitwise_and(hp, one), 6)
            flat6 = rloc * 6
            js = []
            iss = []
            hks = []
            for k in range(6):
                f = flat6 + k
                js.append(lax.shift_right_logical(f, 6))
                iss.append(lax.bitwise_and(f, _C - 1))
                hks.append(lax.shift_left(
                    lax.bitwise_and(
                        lax.shift_right_logical(hp, k + 1), one), 6))

            def dbody(dd, dots):
                dvec = jnp.full((16,), 0, dtype=jnp.int32) + dd * 4
                wbase = hw + dvec
                ebase = [hks[k] + dvec for k in range(6)]
                new = list(dots)
                for u in range(4):
                    wv = plsc.load_gather(wbuf, [rloc, wbase + u])
                    cv = plsc.load_gather(ebuf, [js[0], iss[0], ebase[0] + u])
                    new[0] = new[0] + wv * cv
                    for k in range(1, 6):
                        nv = plsc.load_gather(ebuf,
                                              [js[k], iss[k], ebase[k] + u])
                        new[k] = new[k] + nv * wv
                return tuple(new)

            zero = jnp.zeros((16,), jnp.float32)
            dots = lax.fori_loop(0, _D // 4, dbody, (zero,) * 6)

            acc_pos = acc_pos + _softplus(-dots[0])
            for k in range(1, 6):
                mvec = plsc.load_gather(mask_v, [bvec, rloc * _NEG + (k - 1)])
                acc_neg = acc_neg + _softplus(dots[k] * mvec)
            return (acc_pos, acc_neg)

        return lax.fori_loop(0, _GROUPS, gbody, accs)

    zero = jnp.zeros((16,), jnp.float32)
    accs = (zero, zero)
    descs = [None, None]
    descs[0] = issue(0, 0)
    for c in range(_NCHUNK):
        buf = c % 2
        if c + 1 < _NCHUNK:
            descs[(c + 1) % 2] = issue(c + 1, (c + 1) % 2)
        for d in descs[buf]:
            d.wait()
        accs = compute_chunk(c, buf, accs)

    out_v[0, :] = accs[0]
    out_v[1, :] = accs[1]
    pltpu.sync_copy(out_v, out_hbm.at[wid])


_mesh = plsc.VectorSubcoreMesh(core_axis_name="c", subcore_axis_name="s",
                               num_cores=_NC, num_subcores=_NS)

_sc_kernel = functools.partial(
    pl.kernel,
    out_type=jax.ShapeDtypeStruct((_NW, 2, 16), jnp.float32),
    mesh=_mesh,
    compiler_params=pltpu.CompilerParams(needs_layout_passes=False,
                                         use_tc_tiling_on_sc=True),
    scratch_types=[
        pltpu.VMEM((_NCHUNK, _C), jnp.int32),
        pltpu.VMEM((_NCHUNK * 6, _C), jnp.int32),
        pltpu.VMEM((_ROWS_PER_TILE,), jnp.int32),
        pltpu.VMEM((2, _NEG * _C), jnp.float32),
        pltpu.VMEM((2, _C, 128), jnp.float32),
        pltpu.VMEM((2, 6, _C, 128), jnp.float32),
        pltpu.VMEM((2, 16), jnp.float32),
        pltpu.SemaphoreType.DMA,
        pltpu.SemaphoreType.DMA,
    ],
)(_sc_body)


_TROWS = 16384
_SPLIT = 30 * _TROWS
_RPAD = 1000064
_PROWS = _RPAD - _SPLIT


def _tc_pack_body(lo_ref, hi_ref, out_ref):
    out_ref[...] = jnp.concatenate([lo_ref[...].T, hi_ref[...].T], axis=1)


_tc_pack = pl.pallas_call(
    _tc_pack_body,
    grid=((_PROWS + _TROWS - 1) // _TROWS,),
    in_specs=[pl.BlockSpec((_D, _TROWS), lambda j: (0, j)),
              pl.BlockSpec((_D, _TROWS), lambda j: (0, j + _SPLIT // _TROWS))],
    out_specs=pl.BlockSpec((_TROWS, 2 * _D), lambda j: (j, 0)),
    out_shape=jax.ShapeDtypeStruct((_PROWS, 2 * _D), jnp.float32),
)


def kernel(data, emb0, emb1):
    data = data.astype(jnp.int32)
    widx = data[:, 0]
    e1idx = data[:, 1:2 + _NEG]
    wh = (widx >= _SPLIT).astype(jnp.int32)
    e1h = (e1idx >= _SPLIT).astype(jnp.int32)
    widxp = (widx - wh * _SPLIT).reshape(_NW, _NCHUNK, _C)
    e1idxp = (e1idx - e1h * _SPLIT).reshape(_NW, _NCHUNK * 6, _C)
    hpack = (wh | jnp.left_shift(
        e1h, jnp.arange(1, 7, dtype=jnp.int32)[None, :]).sum(axis=1)
             ).reshape(_NW, _ROWS_PER_TILE)
    maskf = data[:, 2 + _NEG:].astype(jnp.float32).reshape(
        _NW, _NCHUNK, _NEG * _C)
    emb0t = emb0.T
    emb1t = emb1.T
    emb0p = _tc_pack(emb0t, emb0t)
    emb1p = _tc_pack(emb1t, emb1t)
    out = _sc_kernel(widxp, e1idxp, hpack, maskf, emb0p, emb1p)
    pos_loss = jnp.sum(out[:, 0, :])
    neg_loss = jnp.sum(out[:, 1, :])
    return (pos_loss, neg_loss)
